# baseline (speedup 1.0000x reference)
import functools

import jax
import jax.numpy as jnp
from jax import lax
from jax.experimental import pallas as pl
from jax.experimental.pallas import tpu as pltpu
from jax.experimental.pallas import tpu_sc as plsc

NX = 100000
NY = 100000
DX = 32
DY = 32
HIDDEN = 100
HIDDEN_PAD = 128
Q = 1024
BK = 2000
BIG_I32 = 2**30


def _sc_gather(indexes, x_weight):
    info = plsc.get_sparse_core_info()
    nc, ns = info.num_cores, info.num_subcores
    nw = nc * ns
    b_per_w = Q // nw
    mesh = plsc.VectorSubcoreMesh(core_axis_name="c", subcore_axis_name="s")

    @functools.partial(
        pl.kernel,
        out_type=jax.ShapeDtypeStruct((Q, DX), jnp.float32),
        mesh=mesh,
        scratch_types=[
            pltpu.VMEM((b_per_w,), jnp.int32),
            pltpu.VMEM((b_per_w, DX), jnp.float32),
            pltpu.SemaphoreType.DMA,
        ],
        compiler_params=pltpu.CompilerParams(use_tc_tiling_on_sc=False),
    )
    def gather_kernel(idx_hbm, table_hbm, out_hbm, idx_v, rows_v, sem):
        wid = lax.axis_index("s") * nc + lax.axis_index("c")
        base = wid * b_per_w
        pltpu.sync_copy(idx_hbm.at[pl.ds(base, b_per_w)], idx_v)
        pltpu.async_copy(table_hbm.at[idx_v], rows_v, sem).wait()
        pltpu.sync_copy(rows_v, out_hbm.at[pl.ds(base, b_per_w)])

    return gather_kernel(indexes, x_weight)


def _tc_body(gxt_ref, w1t_ref, b1c_ref, w2t_ref, b2c_ref, idx_ref,
             y_ref, out_ref, qt_s, val_s, ind_s, m_s):
    k = pl.program_id(0)

    @pl.when(k == 0)
    def _init():
        h = jnp.dot(w1t_ref[:, :], gxt_ref[:, :],
                    preferred_element_type=jnp.float32) + b1c_ref[:, :]
        h = jnp.maximum(h, 0.0)
        qt_s[0:DY, :] = jnp.dot(w2t_ref[:, :], h,
                                preferred_element_type=jnp.float32) + b2c_ref[:, :]
        qt_s[DY:DY + 8, :] = jnp.ones((8, Q), dtype=jnp.float32)
        val_s[:, :] = jnp.full((1, Q), jnp.inf, dtype=jnp.float32)
        ind_s[:, :] = jnp.zeros((1, Q), dtype=jnp.int32)

    yb = y_ref[:, :]
    tn = jnp.sum(yb * yb, axis=1, keepdims=True)
    yaug = jnp.concatenate(
        [yb * -2.0, tn, jnp.zeros((BK, 7), dtype=jnp.float32)], axis=1)
    m_s[:, :] = jnp.dot(yaug, qt_s[:, :],
                        preferred_element_type=jnp.float32)

    def step(i, carry):
        rmin, rtile = carry
        tile = m_s[pl.ds(i * 8, 8), :]
        lt = tile < rmin
        return (jnp.where(lt, tile, rmin),
                jnp.where(lt, jnp.full((8, Q), i, dtype=jnp.int32), rtile))

    rmin0 = m_s[0:8, :]
    rtile0 = jnp.zeros((8, Q), dtype=jnp.int32)
    rmin, rtile = lax.fori_loop(1, BK // 8, step, (rmin0, rtile0))
    sub = lax.broadcasted_iota(jnp.int32, (8, Q), 0)
    fidx = rtile * 8 + sub
    bmin = jnp.min(rmin, axis=0, keepdims=True)
    bidx = jnp.min(jnp.where(rmin == bmin, fidx, BIG_I32),
                   axis=0, keepdims=True)
    better = bmin < val_s[:, :]
    val_s[:, :] = jnp.where(better, bmin, val_s[:, :])
    ind_s[:, :] = jnp.where(better, bidx + k * BK, ind_s[:, :])

    @pl.when(k == pl.num_programs(0) - 1)
    def _finish():
        mism = (ind_s[:, :] != idx_ref[:, :]).astype(jnp.float32)
        out_ref[0, 0] = jnp.sum(mism) / Q


def _tc_loss(gxt, w1t, b1c, w2t, b2c, idxrow, y_weight, interpret=False):
    grid = NY // BK
    return pl.pallas_call(
        _tc_body,
        grid=(grid,),
        in_specs=[
            pl.BlockSpec((DX, Q), lambda k: (0, 0)),
            pl.BlockSpec((HIDDEN_PAD, DX), lambda k: (0, 0)),
            pl.BlockSpec((HIDDEN_PAD, 1), lambda k: (0, 0)),
            pl.BlockSpec((DY, HIDDEN_PAD), lambda k: (0, 0)),
            pl.BlockSpec((DY, 1), lambda k: (0, 0)),
            pl.BlockSpec((1, Q), lambda k: (0, 0)),
            pl.BlockSpec((BK, DY), lambda k: (k, 0)),
        ],
        out_specs=pl.BlockSpec(memory_space=pltpu.SMEM),
        out_shape=jax.ShapeDtypeStruct((1, 1), jnp.float32),
        scratch_shapes=[
            pltpu.VMEM((DY + 8, Q), jnp.float32),
            pltpu.VMEM((1, Q), jnp.float32),
            pltpu.VMEM((1, Q), jnp.int32),
            pltpu.VMEM((BK, Q), jnp.float32),
        ],
        compiler_params=pltpu.CompilerParams(
            dimension_semantics=("arbitrary",),
        ),
        interpret=interpret,
    )(gxt, w1t, b1c, w2t, b2c, idxrow, y_weight)


def kernel(x_weight, y_weight, W1, b1, W2, b2, indexes):
    gx = _sc_gather(indexes, x_weight)
    gxt = gx.T
    w1t = jnp.zeros((HIDDEN_PAD, DX), jnp.float32).at[:HIDDEN, :].set(W1.T)
    b1c = jnp.zeros((HIDDEN_PAD, 1), jnp.float32).at[:HIDDEN, 0].set(b1)
    w2t = jnp.zeros((DY, HIDDEN_PAD), jnp.float32).at[:, :HIDDEN].set(W2.T)
    b2c = b2.reshape(DY, 1)
    idxrow = indexes.reshape(1, Q)
    out = _tc_loss(gxt, w1t, b1c, w2t, b2c, idxrow, y_weight)
    return out[0, 0]


# speedup vs baseline: 1.2833x; 1.2833x over previous
import functools

import jax
import jax.numpy as jnp
from jax import lax
from jax.experimental import pallas as pl
from jax.experimental.pallas import tpu as pltpu
from jax.experimental.pallas import tpu_sc as plsc

NX = 100000
NY = 100000
DX = 32
DY = 32
HIDDEN = 100
HIDDEN_PAD = 128
Q = 1024
BK = 2000
BIG_I32 = 2**30


def _sc_gather(indexes, x_weight):
    info = plsc.get_sparse_core_info()
    nc, ns = info.num_cores, info.num_subcores
    nw = nc * ns
    b_per_w = Q // nw
    mesh = plsc.VectorSubcoreMesh(core_axis_name="c", subcore_axis_name="s")

    @functools.partial(
        pl.kernel,
        out_type=jax.ShapeDtypeStruct((Q, DX), jnp.float32),
        mesh=mesh,
        scratch_types=[
            pltpu.VMEM((b_per_w,), jnp.int32),
            pltpu.VMEM((b_per_w, DX), jnp.float32),
            pltpu.SemaphoreType.DMA,
        ],
        compiler_params=pltpu.CompilerParams(use_tc_tiling_on_sc=False),
    )
    def gather_kernel(idx_hbm, table_hbm, out_hbm, idx_v, rows_v, sem):
        wid = lax.axis_index("s") * nc + lax.axis_index("c")
        base = wid * b_per_w
        pltpu.sync_copy(idx_hbm.at[pl.ds(base, b_per_w)], idx_v)
        pltpu.async_copy(table_hbm.at[idx_v], rows_v, sem).wait()
        pltpu.sync_copy(rows_v, out_hbm.at[pl.ds(base, b_per_w)])

    return gather_kernel(indexes, x_weight)


def _tc_body(gxt_ref, w1t_ref, b1c_ref, w2t_ref, b2c_ref, idx_ref,
             y_ref, out_ref, qt_s, val_s, ind_s):
    k = pl.program_id(0)

    @pl.when(k == 0)
    def _init():
        h = jnp.dot(w1t_ref[:, :], gxt_ref[:, :],
                    preferred_element_type=jnp.float32) + b1c_ref[:, :]
        h = jnp.maximum(h, 0.0)
        qt_s[0:DY, :] = jnp.dot(w2t_ref[:, :], h,
                                preferred_element_type=jnp.float32) + b2c_ref[:, :]
        qt_s[DY:DY + 8, :] = jnp.ones((8, Q), dtype=jnp.float32)
        val_s[:, :] = jnp.full((1, Q), jnp.inf, dtype=jnp.float32)
        ind_s[:, :] = jnp.zeros((1, Q), dtype=jnp.int32)

    yb = y_ref[:, :]
    tn = jnp.sum(yb * yb, axis=1, keepdims=True)
    yaug = jnp.concatenate(
        [yb * -2.0, tn, jnp.zeros((BK, 7), dtype=jnp.float32)], axis=1)
    m = jnp.dot(yaug, qt_s[:, :],
                preferred_element_type=jnp.float32)
    bmin = jnp.min(m, axis=0, keepdims=True)
    iota = lax.broadcasted_iota(jnp.int32, m.shape, 0)
    bidx = jnp.min(jnp.where(m == bmin, iota, BIG_I32),
                   axis=0, keepdims=True)
    better = bmin < val_s[:, :]
    val_s[:, :] = jnp.where(better, bmin, val_s[:, :])
    ind_s[:, :] = jnp.where(better, bidx + k * BK, ind_s[:, :])

    @pl.when(k == pl.num_programs(0) - 1)
    def _finish():
        mism = (ind_s[:, :] != idx_ref[:, :]).astype(jnp.float32)
        out_ref[0, 0] = jnp.sum(mism) / Q


def _tc_loss(gxt, w1t, b1c, w2t, b2c, idxrow, y_weight, interpret=False):
    grid = NY // BK
    return pl.pallas_call(
        _tc_body,
        grid=(grid,),
        in_specs=[
            pl.BlockSpec((DX, Q), lambda k: (0, 0)),
            pl.BlockSpec((HIDDEN_PAD, DX), lambda k: (0, 0)),
            pl.BlockSpec((HIDDEN_PAD, 1), lambda k: (0, 0)),
            pl.BlockSpec((DY, HIDDEN_PAD), lambda k: (0, 0)),
            pl.BlockSpec((DY, 1), lambda k: (0, 0)),
            pl.BlockSpec((1, Q), lambda k: (0, 0)),
            pl.BlockSpec((BK, DY), lambda k: (k, 0)),
        ],
        out_specs=pl.BlockSpec(memory_space=pltpu.SMEM),
        out_shape=jax.ShapeDtypeStruct((1, 1), jnp.float32),
        scratch_shapes=[
            pltpu.VMEM((DY + 8, Q), jnp.float32),
            pltpu.VMEM((1, Q), jnp.float32),
            pltpu.VMEM((1, Q), jnp.int32),
        ],
        compiler_params=pltpu.CompilerParams(
            dimension_semantics=("arbitrary",),
        ),
        interpret=interpret,
    )(gxt, w1t, b1c, w2t, b2c, idxrow, y_weight)


def kernel(x_weight, y_weight, W1, b1, W2, b2, indexes):
    gx = _sc_gather(indexes, x_weight)
    gxt = gx.T
    w1t = jnp.zeros((HIDDEN_PAD, DX), jnp.float32).at[:HIDDEN, :].set(W1.T)
    b1c = jnp.zeros((HIDDEN_PAD, 1), jnp.float32).at[:HIDDEN, 0].set(b1)
    w2t = jnp.zeros((DY, HIDDEN_PAD), jnp.float32).at[:, :HIDDEN].set(W2.T)
    b2c = b2.reshape(DY, 1)
    idxrow = indexes.reshape(1, Q)
    out = _tc_loss(gxt, w1t, b1c, w2t, b2c, idxrow, y_weight)
    return out[0, 0]


# speedup vs baseline: 1.4106x; 1.0992x over previous
import functools

import jax
import jax.numpy as jnp
from jax import lax
from jax.experimental import pallas as pl
from jax.experimental.pallas import tpu as pltpu
from jax.experimental.pallas import tpu_sc as plsc

NX = 100000
NY = 100000
DX = 32
DY = 32
HIDDEN = 100
HIDDEN_PAD = 128
Q = 1024
AUG = DY + 8
BKL = 2048
NFULL = NY // BKL
TAIL = NY - NFULL * BKL
GRID = NFULL + 1


def _sc_gather(indexes, x_weight):
    info = plsc.get_sparse_core_info()
    nc, ns = info.num_cores, info.num_subcores
    nw = nc * ns
    b_per_w = Q // nw
    mesh = plsc.VectorSubcoreMesh(core_axis_name="c", subcore_axis_name="s")

    @functools.partial(
        pl.kernel,
        out_type=jax.ShapeDtypeStruct((Q, DX), jnp.float32),
        mesh=mesh,
        scratch_types=[
            pltpu.VMEM((b_per_w,), jnp.int32),
            pltpu.VMEM((b_per_w, DX), jnp.float32),
            pltpu.SemaphoreType.DMA,
        ],
        compiler_params=pltpu.CompilerParams(use_tc_tiling_on_sc=False),
    )
    def gather_kernel(idx_hbm, table_hbm, out_hbm, idx_v, rows_v, sem):
        wid = lax.axis_index("s") * nc + lax.axis_index("c")
        base = wid * b_per_w
        pltpu.sync_copy(idx_hbm.at[pl.ds(base, b_per_w)], idx_v)
        pltpu.async_copy(table_hbm.at[idx_v], rows_v, sem).wait()
        pltpu.sync_copy(rows_v, out_hbm.at[pl.ds(base, b_per_w)])

    return gather_kernel(indexes, x_weight)


def _tc_body(gxt_ref, w1t_ref, b1c_ref, w2t_ref, b2c_ref, idx_ref, yt_ref,
             out_ref, qt_s, ya_s, val_s, ind_s):
    k = pl.program_id(0)

    @pl.when(k == 0)
    def _init():
        h = jnp.dot(w1t_ref[:, :], gxt_ref[:, :],
                    preferred_element_type=jnp.float32) + b1c_ref[:, :]
        h = jnp.maximum(h, 0.0)
        qt_s[0:DY, :] = jnp.dot(w2t_ref[:, :], h,
                                preferred_element_type=jnp.float32) + b2c_ref[:, :]
        rows = lax.broadcasted_iota(jnp.int32, (8, Q), 0)
        qt_s[DY:AUG, :] = jnp.where(rows == 0, 1.0, 0.0)
        val_s[:, :] = jnp.full((1, Q), jnp.inf, dtype=jnp.float32)
        ind_s[:, :] = jnp.zeros((1, Q), dtype=jnp.int32)

    ytb = yt_ref[:, :]
    ya_s[0:DY, :] = ytb * -2.0
    tn = jnp.sum(ytb * ytb, axis=0, keepdims=True)
    ya_s[DY:AUG, :] = jnp.broadcast_to(tn, (8, BKL))
    m = lax.dot_general(ya_s[:, :], qt_s[:, :],
                        (((0,), (0,)), ((), ())),
                        preferred_element_type=jnp.float32)

    def upd(mm):
        bmin = jnp.min(mm, axis=0, keepdims=True)
        iota_f = lax.broadcasted_iota(jnp.int32, mm.shape, 0).astype(jnp.float32)
        bidx_f = jnp.min(jnp.where(mm == bmin, iota_f, 3.0e9),
                         axis=0, keepdims=True)
        better = bmin < val_s[:, :]
        val_s[:, :] = jnp.where(better, bmin, val_s[:, :])
        gidx = bidx_f.astype(jnp.int32) + k * BKL
        ind_s[:, :] = jnp.where(better, gidx, ind_s[:, :])

    @pl.when(k < NFULL)
    def _full():
        upd(m)

    @pl.when(k == NFULL)
    def _tail():
        upd(m[0:TAIL, :])

    @pl.when(k == pl.num_programs(0) - 1)
    def _finish():
        mism = (ind_s[:, :] != idx_ref[:, :]).astype(jnp.float32)
        out_ref[0, 0] = jnp.sum(mism) / Q


def _tc_loss(gxt, w1t, b1c, w2t, b2c, idxrow, yt, interpret=False):
    return pl.pallas_call(
        _tc_body,
        grid=(GRID,),
        in_specs=[
            pl.BlockSpec((DX, Q), lambda k: (0, 0)),
            pl.BlockSpec((HIDDEN_PAD, DX), lambda k: (0, 0)),
            pl.BlockSpec((HIDDEN_PAD, 1), lambda k: (0, 0)),
            pl.BlockSpec((DY, HIDDEN_PAD), lambda k: (0, 0)),
            pl.BlockSpec((DY, 1), lambda k: (0, 0)),
            pl.BlockSpec((1, Q), lambda k: (0, 0)),
            pl.BlockSpec((DY, BKL), lambda k: (0, k)),
        ],
        out_specs=pl.BlockSpec(memory_space=pltpu.SMEM),
        out_shape=jax.ShapeDtypeStruct((1, 1), jnp.float32),
        scratch_shapes=[
            pltpu.VMEM((AUG, Q), jnp.float32),
            pltpu.VMEM((AUG, BKL), jnp.float32),
            pltpu.VMEM((1, Q), jnp.float32),
            pltpu.VMEM((1, Q), jnp.int32),
        ],
        compiler_params=pltpu.CompilerParams(
            dimension_semantics=("arbitrary",),
        ),
        interpret=interpret,
    )(gxt, w1t, b1c, w2t, b2c, idxrow, yt)


def kernel(x_weight, y_weight, W1, b1, W2, b2, indexes):
    gx = _sc_gather(indexes, x_weight)
    gxt = gx.T
    yt = y_weight.T
    w1t = jnp.zeros((HIDDEN_PAD, DX), jnp.float32).at[:HIDDEN, :].set(W1.T)
    b1c = jnp.zeros((HIDDEN_PAD, 1), jnp.float32).at[:HIDDEN, 0].set(b1)
    w2t = jnp.zeros((DY, HIDDEN_PAD), jnp.float32).at[:, :HIDDEN].set(W2.T)
    b2c = b2.reshape(DY, 1)
    idxrow = indexes.reshape(1, Q)
    out = _tc_loss(gxt, w1t, b1c, w2t, b2c, idxrow, yt)
    return out[0, 0]


# speedup vs baseline: 1.4482x; 1.0267x over previous
import functools

import jax
import jax.numpy as jnp
from jax import lax
from jax.experimental import pallas as pl
from jax.experimental.pallas import tpu as pltpu
from jax.experimental.pallas import tpu_sc as plsc

NX = 100000
NY = 100000
DX = 32
DY = 32
HIDDEN = 100
HIDDEN_PAD = 128
Q = 1024
AUG = DY + 8
BKL = 2048
NFULL = NY // BKL
TAIL = NY - NFULL * BKL
GRID = NFULL + 1


def _sc_gather(indexes, x_weight):
    info = plsc.get_sparse_core_info()
    nc, ns = info.num_cores, info.num_subcores
    nw = nc * ns
    b_per_w = Q // nw
    mesh = plsc.VectorSubcoreMesh(core_axis_name="c", subcore_axis_name="s")

    @functools.partial(
        pl.kernel,
        out_type=jax.ShapeDtypeStruct((Q, DX), jnp.float32),
        mesh=mesh,
        scratch_types=[
            pltpu.VMEM((b_per_w,), jnp.int32),
            pltpu.VMEM((b_per_w, DX), jnp.float32),
            pltpu.SemaphoreType.DMA,
        ],
        compiler_params=pltpu.CompilerParams(use_tc_tiling_on_sc=False),
    )
    def gather_kernel(idx_hbm, table_hbm, out_hbm, idx_v, rows_v, sem):
        wid = lax.axis_index("s") * nc + lax.axis_index("c")
        base = wid * b_per_w
        pltpu.sync_copy(idx_hbm.at[pl.ds(base, b_per_w)], idx_v)
        pltpu.async_copy(table_hbm.at[idx_v], rows_v, sem).wait()
        pltpu.sync_copy(rows_v, out_hbm.at[pl.ds(base, b_per_w)])

    return gather_kernel(indexes, x_weight)


def _tc_body(gxt_ref, w1t_ref, b1c_ref, w2t_ref, b2c_ref, idx_ref, yt_ref,
             out_ref, qt_s, ya_s, rowi_s, val_s, ind_s):
    k = pl.program_id(0)

    @pl.when(k == 0)
    def _init():
        h = jnp.dot(w1t_ref[:, :], gxt_ref[:, :],
                    preferred_element_type=jnp.float32) + b1c_ref[:, :]
        h = jnp.maximum(h, 0.0)
        qt_s[0:DY, :] = jnp.dot(w2t_ref[:, :], h,
                                preferred_element_type=jnp.float32) + b2c_ref[:, :]
        rows = lax.broadcasted_iota(jnp.int32, (8, Q), 0)
        qt_s[DY:AUG, :] = jnp.where(rows == 0, 1.0, 0.0)
        rowi_s[:, :] = lax.broadcasted_iota(
            jnp.int32, (1, BKL), 1).astype(jnp.float32)
        val_s[:, :] = jnp.full((1, Q), jnp.inf, dtype=jnp.float32)
        ind_s[:, :] = jnp.zeros((1, Q), dtype=jnp.int32)

    ytb = yt_ref[:, :]
    ya_s[0:DY, :] = ytb * -2.0
    tn = jnp.sum(ytb * ytb, axis=0, keepdims=True)
    ya_s[DY:AUG, :] = jnp.broadcast_to(tn, (8, BKL))
    m = lax.dot_general(ya_s[:, :], qt_s[:, :],
                        (((0,), (0,)), ((), ())),
                        preferred_element_type=jnp.float32)

    def upd(mm):
        bmin = jnp.min(mm, axis=0, keepdims=True)
        maskf = jnp.where(m == bmin, 1.0, 0.0)
        bidx_f = jnp.dot(rowi_s[:, :], maskf,
                         preferred_element_type=jnp.float32)
        better = bmin < val_s[:, :]
        val_s[:, :] = jnp.where(better, bmin, val_s[:, :])
        gidx = bidx_f.astype(jnp.int32) + k * BKL
        ind_s[:, :] = jnp.where(better, gidx, ind_s[:, :])

    @pl.when(k < NFULL)
    def _full():
        upd(m)

    @pl.when(k == NFULL)
    def _tail():
        upd(m[0:TAIL, :])

    @pl.when(k == pl.num_programs(0) - 1)
    def _finish():
        mism = (ind_s[:, :] != idx_ref[:, :]).astype(jnp.float32)
        out_ref[0, 0] = jnp.sum(mism) / Q


def _tc_loss(gxt, w1t, b1c, w2t, b2c, idxrow, yt, interpret=False):
    return pl.pallas_call(
        _tc_body,
        grid=(GRID,),
        in_specs=[
            pl.BlockSpec((DX, Q), lambda k: (0, 0)),
            pl.BlockSpec((HIDDEN_PAD, DX), lambda k: (0, 0)),
            pl.BlockSpec((HIDDEN_PAD, 1), lambda k: (0, 0)),
            pl.BlockSpec((DY, HIDDEN_PAD), lambda k: (0, 0)),
            pl.BlockSpec((DY, 1), lambda k: (0, 0)),
            pl.BlockSpec((1, Q), lambda k: (0, 0)),
            pl.BlockSpec((DY, BKL), lambda k: (0, k)),
        ],
        out_specs=pl.BlockSpec(memory_space=pltpu.SMEM),
        out_shape=jax.ShapeDtypeStruct((1, 1), jnp.float32),
        scratch_shapes=[
            pltpu.VMEM((AUG, Q), jnp.float32),
            pltpu.VMEM((AUG, BKL), jnp.float32),
            pltpu.VMEM((1, BKL), jnp.float32),
            pltpu.VMEM((1, Q), jnp.float32),
            pltpu.VMEM((1, Q), jnp.int32),
        ],
        compiler_params=pltpu.CompilerParams(
            dimension_semantics=("arbitrary",),
        ),
        interpret=interpret,
    )(gxt, w1t, b1c, w2t, b2c, idxrow, yt)


def kernel(x_weight, y_weight, W1, b1, W2, b2, indexes):
    gx = _sc_gather(indexes, x_weight)
    gxt = gx.T
    yt = y_weight.T
    w1t = jnp.zeros((HIDDEN_PAD, DX), jnp.float32).at[:HIDDEN, :].set(W1.T)
    b1c = jnp.zeros((HIDDEN_PAD, 1), jnp.float32).at[:HIDDEN, 0].set(b1)
    w2t = jnp.zeros((DY, HIDDEN_PAD), jnp.float32).at[:, :HIDDEN].set(W2.T)
    b2c = b2.reshape(DY, 1)
    idxrow = indexes.reshape(1, Q)
    out = _tc_loss(gxt, w1t, b1c, w2t, b2c, idxrow, yt)
    return out[0, 0]


# speedup vs baseline: 1.5002x; 1.0359x over previous
import functools

import jax
import jax.numpy as jnp
from jax import lax
from jax.experimental import pallas as pl
from jax.experimental.pallas import tpu as pltpu
from jax.experimental.pallas import tpu_sc as plsc

NX = 100000
NY = 100000
DX = 32
DY = 32
HIDDEN = 100
HIDDEN_PAD = 128
Q = 1024
AUG = DY + 8
BKL = 4096
NFULL = NY // BKL
TAIL = NY - NFULL * BKL
GRID = NFULL + 1


def _sc_gather(indexes, x_weight):
    info = plsc.get_sparse_core_info()
    nc, ns = info.num_cores, info.num_subcores
    nw = nc * ns
    b_per_w = Q // nw
    mesh = plsc.VectorSubcoreMesh(core_axis_name="c", subcore_axis_name="s")

    @functools.partial(
        pl.kernel,
        out_type=jax.ShapeDtypeStruct((Q, DX), jnp.float32),
        mesh=mesh,
        scratch_types=[
            pltpu.VMEM((b_per_w,), jnp.int32),
            pltpu.VMEM((b_per_w, DX), jnp.float32),
            pltpu.SemaphoreType.DMA,
        ],
        compiler_params=pltpu.CompilerParams(use_tc_tiling_on_sc=False),
    )
    def gather_kernel(idx_hbm, table_hbm, out_hbm, idx_v, rows_v, sem):
        wid = lax.axis_index("s") * nc + lax.axis_index("c")
        base = wid * b_per_w
        pltpu.sync_copy(idx_hbm.at[pl.ds(base, b_per_w)], idx_v)
        pltpu.async_copy(table_hbm.at[idx_v], rows_v, sem).wait()
        pltpu.sync_copy(rows_v, out_hbm.at[pl.ds(base, b_per_w)])

    return gather_kernel(indexes, x_weight)


def _tc_body(gxt_ref, w1t_ref, b1c_ref, w2t_ref, b2c_ref, idx_ref, yt_ref,
             out_ref, qt_s, ya_s, rowi_s, val_s, ind_s):
    k = pl.program_id(0)

    @pl.when(k == 0)
    def _init():
        h = jnp.dot(w1t_ref[:, :], gxt_ref[:, :],
                    preferred_element_type=jnp.float32) + b1c_ref[:, :]
        h = jnp.maximum(h, 0.0)
        qt_s[0:DY, :] = jnp.dot(w2t_ref[:, :], h,
                                preferred_element_type=jnp.float32) + b2c_ref[:, :]
        rows = lax.broadcasted_iota(jnp.int32, (8, Q), 0)
        qt_s[DY:AUG, :] = jnp.where(rows == 0, 1.0, 0.0)
        rowi_s[:, :] = lax.broadcasted_iota(
            jnp.int32, (1, BKL), 1).astype(jnp.float32)
        val_s[:, :] = jnp.full((1, Q), jnp.inf, dtype=jnp.float32)
        ind_s[:, :] = jnp.zeros((1, Q), dtype=jnp.int32)

    ytb = yt_ref[:, :]
    ya_s[0:DY, :] = ytb * -2.0
    tn = jnp.sum(ytb * ytb, axis=0, keepdims=True)
    ya_s[DY:AUG, :] = jnp.broadcast_to(tn, (8, BKL))
    m = lax.dot_general(ya_s[:, :], qt_s[:, :],
                        (((0,), (0,)), ((), ())),
                        preferred_element_type=jnp.float32)

    def upd(mm):
        bmin = jnp.min(mm, axis=0, keepdims=True)
        maskf = jnp.where(m == bmin, 1.0, 0.0)
        bidx_f = jnp.dot(rowi_s[:, :], maskf,
                         preferred_element_type=jnp.float32)
        better = bmin < val_s[:, :]
        val_s[:, :] = jnp.where(better, bmin, val_s[:, :])
        gidx = bidx_f.astype(jnp.int32) + k * BKL
        ind_s[:, :] = jnp.where(better, gidx, ind_s[:, :])

    @pl.when(k < NFULL)
    def _full():
        upd(m)

    @pl.when(k == NFULL)
    def _tail():
        upd(m[0:TAIL, :])

    @pl.when(k == pl.num_programs(0) - 1)
    def _finish():
        mism = (ind_s[:, :] != idx_ref[:, :]).astype(jnp.float32)
        out_ref[0, 0] = jnp.sum(mism) / Q


def _tc_loss(gxt, w1t, b1c, w2t, b2c, idxrow, yt, interpret=False):
    return pl.pallas_call(
        _tc_body,
        grid=(GRID,),
        in_specs=[
            pl.BlockSpec((DX, Q), lambda k: (0, 0)),
            pl.BlockSpec((HIDDEN_PAD, DX), lambda k: (0, 0)),
            pl.BlockSpec((HIDDEN_PAD, 1), lambda k: (0, 0)),
            pl.BlockSpec((DY, HIDDEN_PAD), lambda k: (0, 0)),
            pl.BlockSpec((DY, 1), lambda k: (0, 0)),
            pl.BlockSpec((1, Q), lambda k: (0, 0)),
            pl.BlockSpec((DY, BKL), lambda k: (0, k)),
        ],
        out_specs=pl.BlockSpec(memory_space=pltpu.SMEM),
        out_shape=jax.ShapeDtypeStruct((1, 1), jnp.float32),
        scratch_shapes=[
            pltpu.VMEM((AUG, Q), jnp.float32),
            pltpu.VMEM((AUG, BKL), jnp.float32),
            pltpu.VMEM((1, BKL), jnp.float32),
            pltpu.VMEM((1, Q), jnp.float32),
            pltpu.VMEM((1, Q), jnp.int32),
        ],
        compiler_params=pltpu.CompilerParams(
            dimension_semantics=("arbitrary",),
        ),
        interpret=interpret,
    )(gxt, w1t, b1c, w2t, b2c, idxrow, yt)


def kernel(x_weight, y_weight, W1, b1, W2, b2, indexes):
    gx = _sc_gather(indexes, x_weight)
    gxt = gx.T
    yt = y_weight.T
    w1t = jnp.zeros((HIDDEN_PAD, DX), jnp.float32).at[:HIDDEN, :].set(W1.T)
    b1c = jnp.zeros((HIDDEN_PAD, 1), jnp.float32).at[:HIDDEN, 0].set(b1)
    w2t = jnp.zeros((DY, HIDDEN_PAD), jnp.float32).at[:, :HIDDEN].set(W2.T)
    b2c = b2.reshape(DY, 1)
    idxrow = indexes.reshape(1, Q)
    out = _tc_loss(gxt, w1t, b1c, w2t, b2c, idxrow, yt)
    return out[0, 0]


# speedup vs baseline: 1.9304x; 1.2868x over previous
import functools

import jax
import jax.numpy as jnp
from jax import lax
from jax.experimental import pallas as pl
from jax.experimental.pallas import tpu as pltpu
from jax.experimental.pallas import tpu_sc as plsc

NX = 100000
NY = 100000
DX = 32
DY = 32
HIDDEN = 100
HIDDEN_PAD = 128
Q = 1024
AUG = DY + 8
BKL = 4096
NFULL = NY // BKL
TAIL = NY - NFULL * BKL
GRID = NFULL + 1


RING = 8


def _sc_gather(indexes, xt):
    info = plsc.get_sparse_core_info()
    nc, ns = info.num_cores, info.num_subcores
    nw = nc * ns
    b_per_w = Q // nw
    mesh = plsc.VectorSubcoreMesh(core_axis_name="c", subcore_axis_name="s")

    @functools.partial(
        pl.kernel,
        out_type=jax.ShapeDtypeStruct((Q, DX), jnp.float32),
        mesh=mesh,
        scratch_types=[
            pltpu.VMEM((b_per_w,), jnp.int32),
            pltpu.VMEM((RING, DX, 128), jnp.float32),
            pltpu.VMEM((b_per_w, DX), jnp.float32),
            pltpu.SemaphoreType.DMA,
        ],
        compiler_params=pltpu.CompilerParams(use_tc_tiling_on_sc=True,
                                             needs_layout_passes=False),
    )
    def gather_kernel(idx_hbm, xt_hbm, out_hbm, idx_v, ring_v, cols_v, sem):
        wid = lax.axis_index("s") * nc + lax.axis_index("c")
        base = wid * b_per_w
        pltpu.sync_copy(idx_hbm.at[pl.ds(base, b_per_w)], idx_v)
        r0, r1 = lax.iota(jnp.int32, 16), lax.iota(jnp.int32, 16) + 16

        def scal(j):
            chunk = idx_v[pl.ds((j // 16) * 16, 16)]
            msk = lax.iota(jnp.int32, 16) == (j % 16)
            return jnp.sum(jnp.where(msk, chunk, 0))

        def fire(j):
            tc = scal(j) // 128
            off = pl.multiple_of(tc * 128, 128)
            return pltpu.async_copy(
                xt_hbm.at[:, pl.ds(off, 128)], ring_v.at[j % RING], sem)

        handles = [fire(j) for j in range(RING)]
        for j in range(b_per_w):
            handles[j % RING].wait()
            lane = scal(j) % 128
            l16 = jnp.broadcast_to(lane, (16,)).astype(jnp.int32)
            cols_v[j, 0:16] = plsc.load_gather(ring_v.at[j % RING], [r0, l16])
            cols_v[j, 16:32] = plsc.load_gather(ring_v.at[j % RING], [r1, l16])
            if j + RING < b_per_w:
                handles[j % RING] = fire(j + RING)
        pltpu.sync_copy(cols_v, out_hbm.at[pl.ds(base, b_per_w)])

    return gather_kernel(indexes, xt)


def _tc_body(gxt_ref, w1t_ref, b1c_ref, w2t_ref, b2c_ref, idx_ref, yt_ref,
             out_ref, qt_s, ya_s, rowi_s, val_s, ind_s):
    k = pl.program_id(0)

    @pl.when(k == 0)
    def _init():
        h = jnp.dot(w1t_ref[:, :], gxt_ref[:, :],
                    preferred_element_type=jnp.float32) + b1c_ref[:, :]
        h = jnp.maximum(h, 0.0)
        qt_s[0:DY, :] = jnp.dot(w2t_ref[:, :], h,
                                preferred_element_type=jnp.float32) + b2c_ref[:, :]
        rows = lax.broadcasted_iota(jnp.int32, (8, Q), 0)
        qt_s[DY:AUG, :] = jnp.where(rows == 0, 1.0, 0.0)
        rowi_s[:, :] = lax.broadcasted_iota(
            jnp.int32, (1, BKL), 1).astype(jnp.float32)
        val_s[:, :] = jnp.full((1, Q), jnp.inf, dtype=jnp.float32)
        ind_s[:, :] = jnp.zeros((1, Q), dtype=jnp.int32)

    ytb = yt_ref[:, :]
    ya_s[0:DY, :] = ytb * -2.0
    tn = jnp.sum(ytb * ytb, axis=0, keepdims=True)
    ya_s[DY:AUG, :] = jnp.broadcast_to(tn, (8, BKL))
    m = lax.dot_general(ya_s[:, :], qt_s[:, :],
                        (((0,), (0,)), ((), ())),
                        preferred_element_type=jnp.float32)

    def upd(mm):
        bmin = jnp.min(mm, axis=0, keepdims=True)
        maskf = jnp.where(m == bmin, 1.0, 0.0)
        bidx_f = jnp.dot(rowi_s[:, :], maskf,
                         preferred_element_type=jnp.float32)
        better = bmin < val_s[:, :]
        val_s[:, :] = jnp.where(better, bmin, val_s[:, :])
        gidx = bidx_f.astype(jnp.int32) + k * BKL
        ind_s[:, :] = jnp.where(better, gidx, ind_s[:, :])

    @pl.when(k < NFULL)
    def _full():
        upd(m)

    @pl.when(k == NFULL)
    def _tail():
        upd(m[0:TAIL, :])

    @pl.when(k == pl.num_programs(0) - 1)
    def _finish():
        mism = (ind_s[:, :] != idx_ref[:, :]).astype(jnp.float32)
        out_ref[0, 0] = jnp.sum(mism) / Q


def _tc_loss(gxt, w1t, b1c, w2t, b2c, idxrow, yt, interpret=False):
    return pl.pallas_call(
        _tc_body,
        grid=(GRID,),
        in_specs=[
            pl.BlockSpec((DX, Q), lambda k: (0, 0)),
            pl.BlockSpec((HIDDEN_PAD, DX), lambda k: (0, 0)),
            pl.BlockSpec((HIDDEN_PAD, 1), lambda k: (0, 0)),
            pl.BlockSpec((DY, HIDDEN_PAD), lambda k: (0, 0)),
            pl.BlockSpec((DY, 1), lambda k: (0, 0)),
            pl.BlockSpec((1, Q), lambda k: (0, 0)),
            pl.BlockSpec((DY, BKL), lambda k: (0, k)),
        ],
        out_specs=pl.BlockSpec(memory_space=pltpu.SMEM),
        out_shape=jax.ShapeDtypeStruct((1, 1), jnp.float32),
        scratch_shapes=[
            pltpu.VMEM((AUG, Q), jnp.float32),
            pltpu.VMEM((AUG, BKL), jnp.float32),
            pltpu.VMEM((1, BKL), jnp.float32),
            pltpu.VMEM((1, Q), jnp.float32),
            pltpu.VMEM((1, Q), jnp.int32),
        ],
        compiler_params=pltpu.CompilerParams(
            dimension_semantics=("arbitrary",),
        ),
        interpret=interpret,
    )(gxt, w1t, b1c, w2t, b2c, idxrow, yt)


def kernel(x_weight, y_weight, W1, b1, W2, b2, indexes):
    gx = _sc_gather(indexes, x_weight.T)
    gxt = gx.T
    yt = y_weight.T
    w1t = jnp.zeros((HIDDEN_PAD, DX), jnp.float32).at[:HIDDEN, :].set(W1.T)
    b1c = jnp.zeros((HIDDEN_PAD, 1), jnp.float32).at[:HIDDEN, 0].set(b1)
    w2t = jnp.zeros((DY, HIDDEN_PAD), jnp.float32).at[:, :HIDDEN].set(W2.T)
    b2c = b2.reshape(DY, 1)
    idxrow = indexes.reshape(1, Q)
    out = _tc_loss(gxt, w1t, b1c, w2t, b2c, idxrow, yt)
    return out[0, 0]


# speedup vs baseline: 1.9320x; 1.0008x over previous
import functools

import jax
import jax.numpy as jnp
from jax import lax
from jax.experimental import pallas as pl
from jax.experimental.pallas import tpu as pltpu
from jax.experimental.pallas import tpu_sc as plsc

NX = 100000
NY = 100000
DX = 32
DY = 32
HIDDEN = 100
HIDDEN_PAD = 128
Q = 1024
AUG = DY + 8
BKL = 8192
NFULL = NY // BKL
TAIL = NY - NFULL * BKL
GRID = NFULL + 1


RING = 8


def _sc_gather(indexes, xt):
    info = plsc.get_sparse_core_info()
    nc, ns = info.num_cores, info.num_subcores
    nw = nc * ns
    b_per_w = Q // nw
    mesh = plsc.VectorSubcoreMesh(core_axis_name="c", subcore_axis_name="s")

    @functools.partial(
        pl.kernel,
        out_type=jax.ShapeDtypeStruct((Q, DX), jnp.float32),
        mesh=mesh,
        scratch_types=[
            pltpu.VMEM((b_per_w,), jnp.int32),
            pltpu.VMEM((RING, DX, 128), jnp.float32),
            pltpu.VMEM((b_per_w, DX), jnp.float32),
            pltpu.SemaphoreType.DMA,
        ],
        compiler_params=pltpu.CompilerParams(use_tc_tiling_on_sc=True,
                                             needs_layout_passes=False),
    )
    def gather_kernel(idx_hbm, xt_hbm, out_hbm, idx_v, ring_v, cols_v, sem):
        wid = lax.axis_index("s") * nc + lax.axis_index("c")
        base = wid * b_per_w
        pltpu.sync_copy(idx_hbm.at[pl.ds(base, b_per_w)], idx_v)
        r0, r1 = lax.iota(jnp.int32, 16), lax.iota(jnp.int32, 16) + 16

        def scal(j):
            chunk = idx_v[pl.ds((j // 16) * 16, 16)]
            msk = lax.iota(jnp.int32, 16) == (j % 16)
            return jnp.sum(jnp.where(msk, chunk, 0))

        def fire(j):
            tc = scal(j) // 128
            off = pl.multiple_of(tc * 128, 128)
            return pltpu.async_copy(
                xt_hbm.at[:, pl.ds(off, 128)], ring_v.at[j % RING], sem)

        handles = [fire(j) for j in range(RING)]
        for j in range(b_per_w):
            handles[j % RING].wait()
            lane = scal(j) % 128
            l16 = jnp.broadcast_to(lane, (16,)).astype(jnp.int32)
            cols_v[j, 0:16] = plsc.load_gather(ring_v.at[j % RING], [r0, l16])
            cols_v[j, 16:32] = plsc.load_gather(ring_v.at[j % RING], [r1, l16])
            if j + RING < b_per_w:
                handles[j % RING] = fire(j + RING)
        pltpu.sync_copy(cols_v, out_hbm.at[pl.ds(base, b_per_w)])

    return gather_kernel(indexes, xt)


def _tc_body(gxt_ref, w1t_ref, b1c_ref, w2t_ref, b2c_ref, idx_ref, yt_ref,
             out_ref, qt_s, ya_s, rowi_s, val_s, ind_s):
    k = pl.program_id(0)

    @pl.when(k == 0)
    def _init():
        h = jnp.dot(w1t_ref[:, :], gxt_ref[:, :],
                    preferred_element_type=jnp.float32) + b1c_ref[:, :]
        h = jnp.maximum(h, 0.0)
        qt_s[0:DY, :] = jnp.dot(w2t_ref[:, :], h,
                                preferred_element_type=jnp.float32) + b2c_ref[:, :]
        rows = lax.broadcasted_iota(jnp.int32, (8, Q), 0)
        qt_s[DY:AUG, :] = jnp.where(rows == 0, 1.0, 0.0)
        rowi_s[:, :] = lax.broadcasted_iota(
            jnp.int32, (1, BKL), 1).astype(jnp.float32)
        val_s[:, :] = jnp.full((1, Q), jnp.inf, dtype=jnp.float32)
        ind_s[:, :] = jnp.zeros((1, Q), dtype=jnp.int32)

    ytb = yt_ref[:, :]
    ya_s[0:DY, :] = ytb * -2.0
    tn = jnp.sum(ytb * ytb, axis=0, keepdims=True)
    ya_s[DY:AUG, :] = jnp.broadcast_to(tn, (8, BKL))
    m = lax.dot_general(ya_s[:, :], qt_s[:, :],
                        (((0,), (0,)), ((), ())),
                        preferred_element_type=jnp.float32)

    def upd(mm):
        bmin = jnp.min(mm, axis=0, keepdims=True)
        maskf = jnp.where(m == bmin, 1.0, 0.0)
        bidx_f = jnp.dot(rowi_s[:, :], maskf,
                         preferred_element_type=jnp.float32)
        better = bmin < val_s[:, :]
        val_s[:, :] = jnp.where(better, bmin, val_s[:, :])
        gidx = bidx_f.astype(jnp.int32) + k * BKL
        ind_s[:, :] = jnp.where(better, gidx, ind_s[:, :])

    @pl.when(k < NFULL)
    def _full():
        upd(m)

    @pl.when(k == NFULL)
    def _tail():
        upd(m[0:TAIL, :])

    @pl.when(k == pl.num_programs(0) - 1)
    def _finish():
        mism = (ind_s[:, :] != idx_ref[:, :]).astype(jnp.float32)
        out_ref[0, 0] = jnp.sum(mism) / Q


def _tc_loss(gxt, w1t, b1c, w2t, b2c, idxrow, yt, interpret=False):
    return pl.pallas_call(
        _tc_body,
        grid=(GRID,),
        in_specs=[
            pl.BlockSpec((DX, Q), lambda k: (0, 0)),
            pl.BlockSpec((HIDDEN_PAD, DX), lambda k: (0, 0)),
            pl.BlockSpec((HIDDEN_PAD, 1), lambda k: (0, 0)),
            pl.BlockSpec((DY, HIDDEN_PAD), lambda k: (0, 0)),
            pl.BlockSpec((DY, 1), lambda k: (0, 0)),
            pl.BlockSpec((1, Q), lambda k: (0, 0)),
            pl.BlockSpec((DY, BKL), lambda k: (0, k)),
        ],
        out_specs=pl.BlockSpec(memory_space=pltpu.SMEM),
        out_shape=jax.ShapeDtypeStruct((1, 1), jnp.float32),
        scratch_shapes=[
            pltpu.VMEM((AUG, Q), jnp.float32),
            pltpu.VMEM((AUG, BKL), jnp.float32),
            pltpu.VMEM((1, BKL), jnp.float32),
            pltpu.VMEM((1, Q), jnp.float32),
            pltpu.VMEM((1, Q), jnp.int32),
        ],
        compiler_params=pltpu.CompilerParams(
            dimension_semantics=("arbitrary",),
            vmem_limit_bytes=128 * 1024 * 1024,
        ),
        interpret=interpret,
    )(gxt, w1t, b1c, w2t, b2c, idxrow, yt)


def kernel(x_weight, y_weight, W1, b1, W2, b2, indexes):
    gx = _sc_gather(indexes, x_weight.T)
    gxt = gx.T
    yt = y_weight.T
    w1t = jnp.zeros((HIDDEN_PAD, DX), jnp.float32).at[:HIDDEN, :].set(W1.T)
    b1c = jnp.zeros((HIDDEN_PAD, 1), jnp.float32).at[:HIDDEN, 0].set(b1)
    w2t = jnp.zeros((DY, HIDDEN_PAD), jnp.float32).at[:, :HIDDEN].set(W2.T)
    b2c = b2.reshape(DY, 1)
    idxrow = indexes.reshape(1, Q)
    out = _tc_loss(gxt, w1t, b1c, w2t, b2c, idxrow, yt)
    return out[0, 0]
